# baseline (speedup 1.0000x reference)
import functools

import jax
import jax.numpy as jnp
from jax import lax
from jax.experimental import pallas as pl
from jax.experimental.pallas import tpu as pltpu
from jax.experimental.pallas import tpu_sc as plsc


def _sc_geometry():
    try:
        info = plsc.get_sparse_core_info()
        return info.num_cores, info.num_subcores
    except Exception:
        return 2, 16


def _proj_body(nf_ref, w_ref, o_ref):
    o_ref[...] = jnp.dot(nf_ref[...], w_ref[...],
                         preferred_element_type=jnp.float32)


def _project(node_feat, W_in, block_rows):
    n, d_in = node_feat.shape
    d = W_in.shape[1]
    grid = n // block_rows
    return pl.pallas_call(
        _proj_body,
        grid=(grid,),
        in_specs=[
            pl.BlockSpec((block_rows, d_in), lambda i: (i, 0)),
            pl.BlockSpec((d_in, d), lambda i: (0, 0)),
        ],
        out_specs=pl.BlockSpec((block_rows, d), lambda i: (i, 0)),
        out_shape=jax.ShapeDtypeStruct((n, d), jnp.float32),
    )(node_feat, W_in)


_GROUP = 4


def _sc_aggregate(h, src3, dst3, zeros_pad, n_pad, chunks, chunk):
    d = h.shape[1]
    nc, ns = _sc_geometry()
    rows_per_tile = n_pad // ns
    ngroups = chunks // _GROUP
    mesh = plsc.VectorSubcoreMesh(core_axis_name="c", subcore_axis_name="s")

    @functools.partial(
        pl.kernel,
        mesh=mesh,
        out_type=jax.ShapeDtypeStruct((nc, n_pad, d), jnp.float32),
        scratch_types=[
            pltpu.VMEM((chunks, chunk), jnp.int32),
            pltpu.VMEM((chunks, chunk), jnp.int32),
            pltpu.VMEM((2, _GROUP, chunk, d), jnp.float32),
            pltpu.VMEM_SHARED((n_pad, d), jnp.float32),
            pltpu.SemaphoreType.DMA,
            pltpu.SemaphoreType.DMA,
        ],
        compiler_params=pltpu.CompilerParams(use_tc_tiling_on_sc=False),
    )
    def agg_kernel(h_hbm, src_hbm, dst_hbm, zero_hbm, out_hbm,
                   src_v, dst_v, rows_v, acc_sh, sem_g, sem_s):
        cid = lax.axis_index("c")
        sid = lax.axis_index("s")
        wid = cid * ns + sid

        pltpu.sync_copy(src_hbm.at[wid], src_v)
        pltpu.sync_copy(dst_hbm.at[wid], dst_v)

        row0 = sid * rows_per_tile
        pltpu.sync_copy(zero_hbm.at[pl.ds(row0, rows_per_tile)],
                        acc_sh.at[pl.ds(row0, rows_per_tile)])
        plsc.subcore_barrier()

        def fire_gathers(g, half):
            for b in range(_GROUP):
                pltpu.async_copy(h_hbm.at[src_v.at[g * _GROUP + b]],
                                 rows_v.at[half, b], sem_g)

        def half_step(g, half):
            for b in range(_GROUP):
                pltpu.make_async_copy(h_hbm.at[src_v.at[g * _GROUP + b]],
                                      rows_v.at[half, b], sem_g).wait()

            @pl.when(g + 1 < ngroups)
            def _():
                fire_gathers(g + 1, 1 - half)

            descs = [
                pltpu.async_copy(rows_v.at[half, b],
                                 acc_sh.at[dst_v.at[g * _GROUP + b]],
                                 sem_s, add=True)
                for b in range(_GROUP)
            ]
            for dsc in descs:
                dsc.wait()

        fire_gathers(0, 0)

        def outer(o, _):
            half_step(2 * o, 0)
            half_step(2 * o + 1, 1)
            return ()

        lax.fori_loop(0, ngroups // 2, outer, ())
        plsc.subcore_barrier()

        pltpu.sync_copy(acc_sh.at[pl.ds(row0, rows_per_tile)],
                        out_hbm.at[cid, pl.ds(row0, rows_per_tile)])

    return agg_kernel(h, src3, dst3, zeros_pad)


def _silu(x):
    return x * jax.nn.sigmoid(x)


def _tail_body(n_nodes, parts_ref, h_ref, wmsg_ref, wupd_ref,
               state_ref, wstate_ref, lat_ref, wlat_ref,
               w1_ref, b1_ref, w2_ref, b2_ref, o_ref, acc_ref):
    i = pl.program_id(0)
    nblk = pl.num_programs(0)

    @pl.when(i == 0)
    def _():
        acc_ref[...] = jnp.zeros_like(acc_ref)

    agg0 = parts_ref[0] + parts_ref[1]
    agg = jnp.dot(agg0, wmsg_ref[...], preferred_element_type=jnp.float32)
    h2 = _silu(jnp.dot(h_ref[...], wupd_ref[...],
                       preferred_element_type=jnp.float32) + agg)
    acc_ref[...] += jnp.sum(h2, axis=0, keepdims=True)

    @pl.when(i == nblk - 1)
    def _():
        cond = (jnp.dot(state_ref[...], wstate_ref[...],
                        preferred_element_type=jnp.float32)
                + jnp.dot(lat_ref[...], wlat_ref[...],
                          preferred_element_type=jnp.float32))
        g = acc_ref[...] / float(n_nodes) + cond
        hid = _silu(jnp.dot(g, w1_ref[...],
                            preferred_element_type=jnp.float32) + b1_ref[...])
        o_ref[...] = jnp.dot(hid, w2_ref[...],
                             preferred_element_type=jnp.float32) + b2_ref[...]


def _tail(parts, h, W_msg, W_upd, state_attr, W_state, lat9, W_lat,
          W1, b1, W2, b2, block_rows):
    n, d = h.shape
    hid = W1.shape[1]
    grid = n // block_rows
    full = lambda shape: pl.BlockSpec(shape, lambda i: tuple(0 for _ in shape))
    return pl.pallas_call(
        functools.partial(_tail_body, n),
        grid=(grid,),
        in_specs=[
            pl.BlockSpec((2, block_rows, d), lambda i: (0, i, 0)),
            pl.BlockSpec((block_rows, d), lambda i: (i, 0)),
            full((d, d)),
            full((d, d)),
            full(state_attr.shape),
            full(W_state.shape),
            full(lat9.shape),
            full(W_lat.shape),
            full((d, hid)),
            full((1, hid)),
            full((hid, 1)),
            full((1, 1)),
        ],
        out_specs=pl.BlockSpec((1, 1), lambda i: (0, 0)),
        out_shape=jax.ShapeDtypeStruct((1, 1), jnp.float32),
        scratch_shapes=[pltpu.VMEM((1, d), jnp.float32)],
    )(parts, h, W_msg, W_upd, state_attr, W_state, lat9, W_lat, W1, b1, W2, b2)


def kernel(node_feat, edge_index, lattice, state_attr, W_in, W_msg, W_upd,
           W_state, W_lat, W1, b1, W2, b2):
    n, d_in = node_feat.shape
    d = W_in.shape[1]
    e = edge_index.shape[1]

    nc, ns = _sc_geometry()
    nw = nc * ns
    chunk = 128
    quantum = nw * chunk * 2 * _GROUP
    e_pad = -(-e // quantum) * quantum
    per_w = e_pad // nw
    chunks = per_w // chunk
    n_pad = -(-(n + 1) // 128) * 128

    src = edge_index[0]
    dst = edge_index[1]
    src3 = jnp.concatenate(
        [src, jnp.zeros((e_pad - e,), jnp.int32)]).reshape(nw, chunks, chunk)
    dst3 = jnp.concatenate(
        [dst, jnp.full((e_pad - e,), n, jnp.int32)]).reshape(nw, chunks, chunk)
    zeros_pad = jnp.zeros((n_pad, d), jnp.float32)

    h = _project(node_feat, W_in, block_rows=1000)
    parts = _sc_aggregate(h, src3, dst3, zeros_pad, n_pad, chunks, chunk)

    out = _tail(parts, h, W_msg, W_upd, state_attr, W_state,
                lattice.reshape(1, 9), W_lat, W1,
                b1.reshape(1, -1), W2, b2.reshape(1, 1), block_rows=1000)
    return out.reshape(1)


# speedup vs baseline: 1.3309x; 1.3309x over previous
import functools

import jax
import jax.numpy as jnp
from jax import lax
from jax.experimental import pallas as pl
from jax.experimental.pallas import tpu as pltpu
from jax.experimental.pallas import tpu_sc as plsc


def _sc_geometry():
    try:
        info = plsc.get_sparse_core_info()
        return info.num_cores, info.num_subcores
    except Exception:
        return 2, 16


def _proj_body(nf_ref, w_ref, o_ref, obf_ref):
    h = jnp.dot(nf_ref[...], w_ref[...], preferred_element_type=jnp.float32)
    o_ref[...] = h
    obf_ref[...] = h.astype(jnp.bfloat16)


def _project(node_feat, W_in, block_rows):
    n, d_in = node_feat.shape
    d = W_in.shape[1]
    grid = n // block_rows
    return pl.pallas_call(
        _proj_body,
        grid=(grid,),
        in_specs=[
            pl.BlockSpec((block_rows, d_in), lambda i: (i, 0)),
            pl.BlockSpec((d_in, d), lambda i: (0, 0)),
        ],
        out_specs=[
            pl.BlockSpec((block_rows, d), lambda i: (i, 0)),
            pl.BlockSpec((block_rows, d), lambda i: (i, 0)),
        ],
        out_shape=[
            jax.ShapeDtypeStruct((n, d), jnp.float32),
            jax.ShapeDtypeStruct((n, d), jnp.bfloat16),
        ],
    )(node_feat, W_in)


_GROUP = 4


def _sc_aggregate(h, src3, dst3, zeros_pad, n_pad, chunks, chunk):
    d = h.shape[1]
    nc, ns = _sc_geometry()
    rows_per_tile = n_pad // ns
    ngroups = chunks // _GROUP
    mesh = plsc.VectorSubcoreMesh(core_axis_name="c", subcore_axis_name="s")

    @functools.partial(
        pl.kernel,
        mesh=mesh,
        out_type=jax.ShapeDtypeStruct((nc, n_pad, d), jnp.bfloat16),
        scratch_types=[
            pltpu.VMEM((chunks, chunk), jnp.int32),
            pltpu.VMEM((chunks, chunk), jnp.int32),
            pltpu.VMEM((2, _GROUP, chunk, d), jnp.bfloat16),
            pltpu.VMEM_SHARED((n_pad, d), jnp.bfloat16),
            pltpu.SemaphoreType.DMA,
            pltpu.SemaphoreType.DMA,
        ],
        compiler_params=pltpu.CompilerParams(use_tc_tiling_on_sc=False),
    )
    def agg_kernel(h_hbm, src_hbm, dst_hbm, zero_hbm, out_hbm,
                   src_v, dst_v, rows_v, acc_sh, sem_g, sem_s):
        cid = lax.axis_index("c")
        sid = lax.axis_index("s")
        wid = cid * ns + sid

        pltpu.sync_copy(src_hbm.at[wid], src_v)
        pltpu.sync_copy(dst_hbm.at[wid], dst_v)

        row0 = sid * rows_per_tile
        pltpu.sync_copy(zero_hbm.at[pl.ds(row0, rows_per_tile)],
                        acc_sh.at[pl.ds(row0, rows_per_tile)])
        plsc.subcore_barrier()

        def fire_gathers(g, half):
            for b in range(_GROUP):
                pltpu.async_copy(h_hbm.at[src_v.at[g * _GROUP + b]],
                                 rows_v.at[half, b], sem_g)

        def half_step(g, half):
            for b in range(_GROUP):
                pltpu.make_async_copy(h_hbm.at[src_v.at[g * _GROUP + b]],
                                      rows_v.at[half, b], sem_g).wait()

            @pl.when(g + 1 < ngroups)
            def _():
                fire_gathers(g + 1, 1 - half)

            descs = [
                pltpu.async_copy(rows_v.at[half, b],
                                 acc_sh.at[dst_v.at[g * _GROUP + b]],
                                 sem_s, add=True)
                for b in range(_GROUP)
            ]
            for dsc in descs:
                dsc.wait()

        fire_gathers(0, 0)

        def outer(o, _):
            half_step(2 * o, 0)
            half_step(2 * o + 1, 1)
            return ()

        lax.fori_loop(0, ngroups // 2, outer, ())
        plsc.subcore_barrier()

        pltpu.sync_copy(acc_sh.at[pl.ds(row0, rows_per_tile)],
                        out_hbm.at[cid, pl.ds(row0, rows_per_tile)])

    return agg_kernel(h, src3, dst3, zeros_pad)


def _silu(x):
    return x * jax.nn.sigmoid(x)


def _tail_body(n_nodes, parts_ref, h_ref, wmsg_ref, wupd_ref,
               state_ref, wstate_ref, lat_ref, wlat_ref,
               w1_ref, b1_ref, w2_ref, b2_ref, o_ref, acc_ref):
    i = pl.program_id(0)
    nblk = pl.num_programs(0)

    @pl.when(i == 0)
    def _():
        acc_ref[...] = jnp.zeros_like(acc_ref)

    agg0 = (parts_ref[0].astype(jnp.float32)
            + parts_ref[1].astype(jnp.float32))
    agg = jnp.dot(agg0, wmsg_ref[...], preferred_element_type=jnp.float32)
    h2 = _silu(jnp.dot(h_ref[...], wupd_ref[...],
                       preferred_element_type=jnp.float32) + agg)
    acc_ref[...] += jnp.sum(h2, axis=0, keepdims=True)

    @pl.when(i == nblk - 1)
    def _():
        cond = (jnp.dot(state_ref[...], wstate_ref[...],
                        preferred_element_type=jnp.float32)
                + jnp.dot(lat_ref[...], wlat_ref[...],
                          preferred_element_type=jnp.float32))
        g = acc_ref[...] / float(n_nodes) + cond
        hid = _silu(jnp.dot(g, w1_ref[...],
                            preferred_element_type=jnp.float32) + b1_ref[...])
        o_ref[...] = jnp.dot(hid, w2_ref[...],
                             preferred_element_type=jnp.float32) + b2_ref[...]


def _tail(parts, h, W_msg, W_upd, state_attr, W_state, lat9, W_lat,
          W1, b1, W2, b2, block_rows):
    n, d = h.shape
    hid = W1.shape[1]
    grid = n // block_rows
    full = lambda shape: pl.BlockSpec(shape, lambda i: tuple(0 for _ in shape))
    return pl.pallas_call(
        functools.partial(_tail_body, n),
        grid=(grid,),
        in_specs=[
            pl.BlockSpec((2, block_rows, d), lambda i: (0, i, 0)),
            pl.BlockSpec((block_rows, d), lambda i: (i, 0)),
            full((d, d)),
            full((d, d)),
            full(state_attr.shape),
            full(W_state.shape),
            full(lat9.shape),
            full(W_lat.shape),
            full((d, hid)),
            full((1, hid)),
            full((hid, 1)),
            full((1, 1)),
        ],
        out_specs=pl.BlockSpec((1, 1), lambda i: (0, 0)),
        out_shape=jax.ShapeDtypeStruct((1, 1), jnp.float32),
        scratch_shapes=[pltpu.VMEM((1, d), jnp.float32)],
    )(parts, h, W_msg, W_upd, state_attr, W_state, lat9, W_lat, W1, b1, W2, b2)


def kernel(node_feat, edge_index, lattice, state_attr, W_in, W_msg, W_upd,
           W_state, W_lat, W1, b1, W2, b2):
    n, d_in = node_feat.shape
    d = W_in.shape[1]
    e = edge_index.shape[1]

    nc, ns = _sc_geometry()
    nw = nc * ns
    chunk = 128
    quantum = nw * chunk * 2 * _GROUP
    e_pad = -(-e // quantum) * quantum
    per_w = e_pad // nw
    chunks = per_w // chunk
    n_pad = -(-(n + 1) // 128) * 128

    src = edge_index[0]
    dst = edge_index[1]
    src3 = jnp.concatenate(
        [src, jnp.zeros((e_pad - e,), jnp.int32)]).reshape(nw, chunks, chunk)
    dst3 = jnp.concatenate(
        [dst, jnp.full((e_pad - e,), n, jnp.int32)]).reshape(nw, chunks, chunk)
    zeros_pad = jnp.zeros((n_pad, d), jnp.bfloat16)

    h, h_bf = _project(node_feat, W_in, block_rows=1000)
    parts = _sc_aggregate(h_bf, src3, dst3, zeros_pad, n_pad, chunks, chunk)

    out = _tail(parts, h, W_msg, W_upd, state_attr, W_state,
                lattice.reshape(1, 9), W_lat, W1,
                b1.reshape(1, -1), W2, b2.reshape(1, 1), block_rows=1000)
    return out.reshape(1)


# speedup vs baseline: 2.2969x; 1.7258x over previous
import functools

import jax
import jax.numpy as jnp
from jax import lax
from jax.experimental import pallas as pl
from jax.experimental.pallas import tpu as pltpu
from jax.experimental.pallas import tpu_sc as plsc


def _sc_geometry():
    try:
        info = plsc.get_sparse_core_info()
        return info.num_cores, info.num_subcores
    except Exception:
        return 2, 16


def _proj_body(nf_ref, w_ref, o_ref, obf_ref):
    h = jnp.dot(nf_ref[...], w_ref[...], preferred_element_type=jnp.float32)
    o_ref[...] = h
    obf_ref[...] = h.astype(jnp.bfloat16)


def _project(node_feat, W_in, block_rows):
    n, d_in = node_feat.shape
    d = W_in.shape[1]
    grid = n // block_rows
    return pl.pallas_call(
        _proj_body,
        grid=(grid,),
        in_specs=[
            pl.BlockSpec((block_rows, d_in), lambda i: (i, 0)),
            pl.BlockSpec((d_in, d), lambda i: (0, 0)),
        ],
        out_specs=[
            pl.BlockSpec((block_rows, d), lambda i: (i, 0)),
            pl.BlockSpec((block_rows, d), lambda i: (i, 0)),
        ],
        out_shape=[
            jax.ShapeDtypeStruct((n, d), jnp.float32),
            jax.ShapeDtypeStruct((n, d), jnp.bfloat16),
        ],
    )(node_feat, W_in)


_GROUP = 4


def _sc_aggregate(h, src3, dst3, zeros_pad, n_pad, chunks, chunk):
    d = h.shape[1]
    nc, ns = _sc_geometry()
    rows_per_tile = n_pad // ns
    ngroups = chunks // _GROUP
    mesh = plsc.VectorSubcoreMesh(core_axis_name="c", subcore_axis_name="s")

    @functools.partial(
        pl.kernel,
        mesh=mesh,
        out_type=jax.ShapeDtypeStruct((nc, n_pad, d), jnp.bfloat16),
        scratch_types=[
            pltpu.VMEM((chunks, chunk), jnp.int32),
            pltpu.VMEM((chunks, chunk), jnp.int32),
            pltpu.VMEM((2, _GROUP, chunk, d), jnp.bfloat16),
            pltpu.VMEM_SHARED((n_pad, d), jnp.bfloat16),
            pltpu.VMEM_SHARED((h.shape[0], d), jnp.bfloat16),
            pltpu.SemaphoreType.DMA,
            pltpu.SemaphoreType.DMA,
        ],
        compiler_params=pltpu.CompilerParams(use_tc_tiling_on_sc=False),
    )
    def agg_kernel(h_hbm, src_hbm, dst_hbm, zero_hbm, out_hbm,
                   src_v, dst_v, rows_v, acc_sh, h_sh, sem_g, sem_s):
        cid = lax.axis_index("c")
        sid = lax.axis_index("s")
        wid = cid * ns + sid

        pltpu.sync_copy(src_hbm.at[wid], src_v)
        pltpu.sync_copy(dst_hbm.at[wid], dst_v)

        row0 = sid * rows_per_tile
        h_stripe = h_hbm.shape[0] // ns
        pltpu.sync_copy(h_hbm.at[pl.ds(sid * h_stripe, h_stripe)],
                        h_sh.at[pl.ds(sid * h_stripe, h_stripe)])
        pltpu.sync_copy(zero_hbm.at[pl.ds(row0, rows_per_tile)],
                        acc_sh.at[pl.ds(row0, rows_per_tile)])
        plsc.subcore_barrier()

        def fire_gathers(g, half):
            for b in range(_GROUP):
                pltpu.async_copy(h_sh.at[src_v.at[g * _GROUP + b]],
                                 rows_v.at[half, b], sem_g)

        def half_step(g, half):
            for b in range(_GROUP):
                pltpu.make_async_copy(h_sh.at[src_v.at[g * _GROUP + b]],
                                      rows_v.at[half, b], sem_g).wait()

            @pl.when(g + 1 < ngroups)
            def _():
                fire_gathers(g + 1, 1 - half)

            descs = [
                pltpu.async_copy(rows_v.at[half, b],
                                 acc_sh.at[dst_v.at[g * _GROUP + b]],
                                 sem_s, add=True)
                for b in range(_GROUP)
            ]
            for dsc in descs:
                dsc.wait()

        fire_gathers(0, 0)

        def outer(o, _):
            half_step(2 * o, 0)
            half_step(2 * o + 1, 1)
            return ()

        lax.fori_loop(0, ngroups // 2, outer, ())
        plsc.subcore_barrier()

        pltpu.sync_copy(acc_sh.at[pl.ds(row0, rows_per_tile)],
                        out_hbm.at[cid, pl.ds(row0, rows_per_tile)])

    return agg_kernel(h, src3, dst3, zeros_pad)


def _silu(x):
    return x * jax.nn.sigmoid(x)


def _tail_body(n_nodes, parts_ref, h_ref, wmsg_ref, wupd_ref,
               state_ref, wstate_ref, lat_ref, wlat_ref,
               w1_ref, b1_ref, w2_ref, b2_ref, o_ref, acc_ref):
    i = pl.program_id(0)
    nblk = pl.num_programs(0)

    @pl.when(i == 0)
    def _():
        acc_ref[...] = jnp.zeros_like(acc_ref)

    agg0 = (parts_ref[0].astype(jnp.float32)
            + parts_ref[1].astype(jnp.float32))
    agg = jnp.dot(agg0, wmsg_ref[...], preferred_element_type=jnp.float32)
    h2 = _silu(jnp.dot(h_ref[...], wupd_ref[...],
                       preferred_element_type=jnp.float32) + agg)
    acc_ref[...] += jnp.sum(h2, axis=0, keepdims=True)

    @pl.when(i == nblk - 1)
    def _():
        cond = (jnp.dot(state_ref[...], wstate_ref[...],
                        preferred_element_type=jnp.float32)
                + jnp.dot(lat_ref[...], wlat_ref[...],
                          preferred_element_type=jnp.float32))
        g = acc_ref[...] / float(n_nodes) + cond
        hid = _silu(jnp.dot(g, w1_ref[...],
                            preferred_element_type=jnp.float32) + b1_ref[...])
        o_ref[...] = jnp.dot(hid, w2_ref[...],
                             preferred_element_type=jnp.float32) + b2_ref[...]


def _tail(parts, h, W_msg, W_upd, state_attr, W_state, lat9, W_lat,
          W1, b1, W2, b2, block_rows):
    n, d = h.shape
    hid = W1.shape[1]
    grid = n // block_rows
    full = lambda shape: pl.BlockSpec(shape, lambda i: tuple(0 for _ in shape))
    return pl.pallas_call(
        functools.partial(_tail_body, n),
        grid=(grid,),
        in_specs=[
            pl.BlockSpec((2, block_rows, d), lambda i: (0, i, 0)),
            pl.BlockSpec((block_rows, d), lambda i: (i, 0)),
            full((d, d)),
            full((d, d)),
            full(state_attr.shape),
            full(W_state.shape),
            full(lat9.shape),
            full(W_lat.shape),
            full((d, hid)),
            full((1, hid)),
            full((hid, 1)),
            full((1, 1)),
        ],
        out_specs=pl.BlockSpec((1, 1), lambda i: (0, 0)),
        out_shape=jax.ShapeDtypeStruct((1, 1), jnp.float32),
        scratch_shapes=[pltpu.VMEM((1, d), jnp.float32)],
    )(parts, h, W_msg, W_upd, state_attr, W_state, lat9, W_lat, W1, b1, W2, b2)


def kernel(node_feat, edge_index, lattice, state_attr, W_in, W_msg, W_upd,
           W_state, W_lat, W1, b1, W2, b2):
    n, d_in = node_feat.shape
    d = W_in.shape[1]
    e = edge_index.shape[1]

    nc, ns = _sc_geometry()
    nw = nc * ns
    chunk = 128
    quantum = nw * chunk * 2 * _GROUP
    e_pad = -(-e // quantum) * quantum
    per_w = e_pad // nw
    chunks = per_w // chunk
    n_pad = -(-(n + 1) // 128) * 128

    src = edge_index[0]
    dst = edge_index[1]
    src3 = jnp.concatenate(
        [src, jnp.zeros((e_pad - e,), jnp.int32)]).reshape(nw, chunks, chunk)
    dst3 = jnp.concatenate(
        [dst, jnp.full((e_pad - e,), n, jnp.int32)]).reshape(nw, chunks, chunk)
    zeros_pad = jnp.zeros((n_pad, d), jnp.bfloat16)

    h, h_bf = _project(node_feat, W_in, block_rows=1000)
    parts = _sc_aggregate(h_bf, src3, dst3, zeros_pad, n_pad, chunks, chunk)

    out = _tail(parts, h, W_msg, W_upd, state_attr, W_state,
                lattice.reshape(1, 9), W_lat, W1,
                b1.reshape(1, -1), W2, b2.reshape(1, 1), block_rows=1000)
    return out.reshape(1)


# speedup vs baseline: 2.3181x; 1.0093x over previous
import functools

import jax
import jax.numpy as jnp
from jax import lax
from jax.experimental import pallas as pl
from jax.experimental.pallas import tpu as pltpu
from jax.experimental.pallas import tpu_sc as plsc


def _sc_geometry():
    try:
        info = plsc.get_sparse_core_info()
        return info.num_cores, info.num_subcores
    except Exception:
        return 2, 16


def _proj_body(nf_ref, w_ref, o_ref, obf_ref):
    h = jnp.dot(nf_ref[...], w_ref[...], preferred_element_type=jnp.float32)
    o_ref[...] = h
    obf_ref[...] = h.astype(jnp.bfloat16)


def _project(node_feat, W_in, block_rows):
    n, d_in = node_feat.shape
    d = W_in.shape[1]
    grid = n // block_rows
    return pl.pallas_call(
        _proj_body,
        grid=(grid,),
        in_specs=[
            pl.BlockSpec((block_rows, d_in), lambda i: (i, 0)),
            pl.BlockSpec((d_in, d), lambda i: (0, 0)),
        ],
        out_specs=[
            pl.BlockSpec((block_rows, d), lambda i: (i, 0)),
            pl.BlockSpec((block_rows, d), lambda i: (i, 0)),
        ],
        out_shape=[
            jax.ShapeDtypeStruct((n, d), jnp.float32),
            jax.ShapeDtypeStruct((n, d), jnp.bfloat16),
        ],
    )(node_feat, W_in)


_GROUP = 8


def _sc_aggregate(h, src3, dst3, zeros_pad, n_pad, chunks, chunk):
    d = h.shape[1]
    nc, ns = _sc_geometry()
    rows_per_tile = n_pad // ns
    ngroups = chunks // _GROUP
    mesh = plsc.VectorSubcoreMesh(core_axis_name="c", subcore_axis_name="s")

    @functools.partial(
        pl.kernel,
        mesh=mesh,
        out_type=jax.ShapeDtypeStruct((nc, n_pad, d), jnp.bfloat16),
        scratch_types=[
            pltpu.VMEM((chunks, chunk), jnp.int32),
            pltpu.VMEM((chunks, chunk), jnp.int32),
            pltpu.VMEM((2, _GROUP, chunk, d), jnp.bfloat16),
            pltpu.VMEM_SHARED((n_pad, d), jnp.bfloat16),
            pltpu.VMEM_SHARED((h.shape[0], d), jnp.bfloat16),
            pltpu.SemaphoreType.DMA,
            pltpu.SemaphoreType.DMA,
        ],
        compiler_params=pltpu.CompilerParams(use_tc_tiling_on_sc=False),
    )
    def agg_kernel(h_hbm, src_hbm, dst_hbm, zero_hbm, out_hbm,
                   src_v, dst_v, rows_v, acc_sh, h_sh, sem_g, sem_s):
        cid = lax.axis_index("c")
        sid = lax.axis_index("s")
        wid = cid * ns + sid

        pltpu.sync_copy(src_hbm.at[wid], src_v)
        pltpu.sync_copy(dst_hbm.at[wid], dst_v)

        row0 = sid * rows_per_tile
        h_stripe = h_hbm.shape[0] // ns
        pltpu.sync_copy(h_hbm.at[pl.ds(sid * h_stripe, h_stripe)],
                        h_sh.at[pl.ds(sid * h_stripe, h_stripe)])
        pltpu.sync_copy(zero_hbm.at[pl.ds(row0, rows_per_tile)],
                        acc_sh.at[pl.ds(row0, rows_per_tile)])
        plsc.subcore_barrier()

        def fire_gathers(g, half):
            for b in range(_GROUP):
                pltpu.async_copy(h_sh.at[src_v.at[g * _GROUP + b]],
                                 rows_v.at[half, b], sem_g)

        def drain_scatters(g, half):
            for b in range(_GROUP):
                pltpu.make_async_copy(rows_v.at[half, b],
                                      acc_sh.at[dst_v.at[g * _GROUP + b]],
                                      sem_s).wait()

        def half_step(g, half):
            for b in range(_GROUP):
                pltpu.make_async_copy(h_sh.at[src_v.at[g * _GROUP + b]],
                                      rows_v.at[half, b], sem_g).wait()

            @pl.when(g >= 1)
            def _():
                drain_scatters(g - 1, 1 - half)

            @pl.when(g + 1 < ngroups)
            def _():
                fire_gathers(g + 1, 1 - half)

            for b in range(_GROUP):
                pltpu.async_copy(rows_v.at[half, b],
                                 acc_sh.at[dst_v.at[g * _GROUP + b]],
                                 sem_s, add=True)

        fire_gathers(0, 0)

        def outer(o, _):
            half_step(2 * o, 0)
            half_step(2 * o + 1, 1)
            return ()

        lax.fori_loop(0, ngroups // 2, outer, ())
        drain_scatters(ngroups - 1, 1)
        plsc.subcore_barrier()

        pltpu.sync_copy(acc_sh.at[pl.ds(row0, rows_per_tile)],
                        out_hbm.at[cid, pl.ds(row0, rows_per_tile)])

    return agg_kernel(h, src3, dst3, zeros_pad)


def _silu(x):
    return x * jax.nn.sigmoid(x)


def _tail_body(n_nodes, parts_ref, h_ref, wmsg_ref, wupd_ref,
               state_ref, wstate_ref, lat_ref, wlat_ref,
               w1_ref, b1_ref, w2_ref, b2_ref, o_ref, acc_ref):
    i = pl.program_id(0)
    nblk = pl.num_programs(0)

    @pl.when(i == 0)
    def _():
        acc_ref[...] = jnp.zeros_like(acc_ref)

    agg0 = (parts_ref[0].astype(jnp.float32)
            + parts_ref[1].astype(jnp.float32))
    agg = jnp.dot(agg0, wmsg_ref[...], preferred_element_type=jnp.float32)
    h2 = _silu(jnp.dot(h_ref[...], wupd_ref[...],
                       preferred_element_type=jnp.float32) + agg)
    acc_ref[...] += jnp.sum(h2, axis=0, keepdims=True)

    @pl.when(i == nblk - 1)
    def _():
        cond = (jnp.dot(state_ref[...], wstate_ref[...],
                        preferred_element_type=jnp.float32)
                + jnp.dot(lat_ref[...], wlat_ref[...],
                          preferred_element_type=jnp.float32))
        g = acc_ref[...] / float(n_nodes) + cond
        hid = _silu(jnp.dot(g, w1_ref[...],
                            preferred_element_type=jnp.float32) + b1_ref[...])
        o_ref[...] = jnp.dot(hid, w2_ref[...],
                             preferred_element_type=jnp.float32) + b2_ref[...]


def _tail(parts, h, W_msg, W_upd, state_attr, W_state, lat9, W_lat,
          W1, b1, W2, b2, block_rows):
    n, d = h.shape
    hid = W1.shape[1]
    grid = n // block_rows
    full = lambda shape: pl.BlockSpec(shape, lambda i: tuple(0 for _ in shape))
    return pl.pallas_call(
        functools.partial(_tail_body, n),
        grid=(grid,),
        in_specs=[
            pl.BlockSpec((2, block_rows, d), lambda i: (0, i, 0)),
            pl.BlockSpec((block_rows, d), lambda i: (i, 0)),
            full((d, d)),
            full((d, d)),
            full(state_attr.shape),
            full(W_state.shape),
            full(lat9.shape),
            full(W_lat.shape),
            full((d, hid)),
            full((1, hid)),
            full((hid, 1)),
            full((1, 1)),
        ],
        out_specs=pl.BlockSpec((1, 1), lambda i: (0, 0)),
        out_shape=jax.ShapeDtypeStruct((1, 1), jnp.float32),
        scratch_shapes=[pltpu.VMEM((1, d), jnp.float32)],
    )(parts, h, W_msg, W_upd, state_attr, W_state, lat9, W_lat, W1, b1, W2, b2)


def kernel(node_feat, edge_index, lattice, state_attr, W_in, W_msg, W_upd,
           W_state, W_lat, W1, b1, W2, b2):
    n, d_in = node_feat.shape
    d = W_in.shape[1]
    e = edge_index.shape[1]

    nc, ns = _sc_geometry()
    nw = nc * ns
    chunk = 128
    quantum = nw * chunk * 2 * _GROUP
    e_pad = -(-e // quantum) * quantum
    per_w = e_pad // nw
    chunks = per_w // chunk
    n_pad = -(-(n + 1) // 128) * 128

    src = edge_index[0]
    dst = edge_index[1]
    src3 = jnp.concatenate(
        [src, jnp.zeros((e_pad - e,), jnp.int32)]).reshape(nw, chunks, chunk)
    dst3 = jnp.concatenate(
        [dst, jnp.full((e_pad - e,), n, jnp.int32)]).reshape(nw, chunks, chunk)
    zeros_pad = jnp.zeros((n_pad, d), jnp.bfloat16)

    h, h_bf = _project(node_feat, W_in, block_rows=1000)
    parts = _sc_aggregate(h_bf, src3, dst3, zeros_pad, n_pad, chunks, chunk)

    out = _tail(parts, h, W_msg, W_upd, state_attr, W_state,
                lattice.reshape(1, 9), W_lat, W1,
                b1.reshape(1, -1), W2, b2.reshape(1, 1), block_rows=1000)
    return out.reshape(1)


# speedup vs baseline: 2.4407x; 1.0529x over previous
import functools

import jax
import jax.numpy as jnp
from jax import lax
from jax.experimental import pallas as pl
from jax.experimental.pallas import tpu as pltpu
from jax.experimental.pallas import tpu_sc as plsc


def _sc_geometry():
    try:
        info = plsc.get_sparse_core_info()
        return info.num_cores, info.num_subcores
    except Exception:
        return 2, 16


def _proj_body(nf_ref, w_ref, o_ref, obf_ref):
    h = jnp.dot(nf_ref[...], w_ref[...], preferred_element_type=jnp.float32)
    o_ref[...] = h
    obf_ref[...] = h.astype(jnp.bfloat16)


def _project(node_feat, W_in, block_rows):
    n, d_in = node_feat.shape
    d = W_in.shape[1]
    grid = n // block_rows
    return pl.pallas_call(
        _proj_body,
        grid=(grid,),
        in_specs=[
            pl.BlockSpec((block_rows, d_in), lambda i: (i, 0)),
            pl.BlockSpec((d_in, d), lambda i: (0, 0)),
        ],
        out_specs=[
            pl.BlockSpec((block_rows, d), lambda i: (i, 0)),
            pl.BlockSpec((block_rows, d), lambda i: (i, 0)),
        ],
        out_shape=[
            jax.ShapeDtypeStruct((n, d), jnp.float32),
            jax.ShapeDtypeStruct((n, d), jnp.bfloat16),
        ],
    )(node_feat, W_in)


_GROUP = 8


def _sc_aggregate(h, sd3, zeros_pad, n_pad, chunks, chunk):
    d = h.shape[1]
    nc, ns = _sc_geometry()
    rows_per_tile = n_pad // ns
    ngroups = chunks // _GROUP
    mesh = plsc.VectorSubcoreMesh(core_axis_name="c", subcore_axis_name="s")

    @functools.partial(
        pl.kernel,
        mesh=mesh,
        out_type=jax.ShapeDtypeStruct((nc, n_pad, d), jnp.bfloat16),
        scratch_types=[
            pltpu.VMEM((2, chunks, chunk), jnp.int32),
            pltpu.VMEM((2, _GROUP, chunk, d), jnp.bfloat16),
            pltpu.VMEM_SHARED((n_pad, d), jnp.bfloat16),
            pltpu.VMEM_SHARED((h.shape[0], d), jnp.bfloat16),
            pltpu.SemaphoreType.DMA,
            pltpu.SemaphoreType.DMA,
        ],
        compiler_params=pltpu.CompilerParams(use_tc_tiling_on_sc=False),
    )
    def agg_kernel(h_hbm, sd_hbm, zero_hbm, out_hbm,
                   sd_v, rows_v, acc_sh, h_sh, sem_g, sem_s):
        cid = lax.axis_index("c")
        sid = lax.axis_index("s")
        wid = cid * ns + sid
        src_v = sd_v.at[0]
        dst_v = sd_v.at[1]

        pltpu.sync_copy(sd_hbm.at[wid], sd_v)

        row0 = sid * rows_per_tile
        h_stripe = h_hbm.shape[0] // ns
        pltpu.sync_copy(h_hbm.at[pl.ds(sid * h_stripe, h_stripe)],
                        h_sh.at[pl.ds(sid * h_stripe, h_stripe)])
        pltpu.sync_copy(zero_hbm.at[pl.ds(row0, rows_per_tile)],
                        acc_sh.at[pl.ds(row0, rows_per_tile)])
        plsc.subcore_barrier()

        def fire_gathers(g, half):
            for b in range(_GROUP):
                pltpu.async_copy(h_sh.at[src_v.at[g * _GROUP + b]],
                                 rows_v.at[half, b], sem_g)

        def drain_scatters(g, half):
            for b in range(_GROUP):
                pltpu.make_async_copy(rows_v.at[half, b],
                                      acc_sh.at[dst_v.at[g * _GROUP + b]],
                                      sem_s).wait()

        def half_step(g, half):
            for b in range(_GROUP):
                pltpu.make_async_copy(h_sh.at[src_v.at[g * _GROUP + b]],
                                      rows_v.at[half, b], sem_g).wait()

            @pl.when(g >= 1)
            def _():
                drain_scatters(g - 1, 1 - half)

            @pl.when(g + 1 < ngroups)
            def _():
                fire_gathers(g + 1, 1 - half)

            for b in range(_GROUP):
                pltpu.async_copy(rows_v.at[half, b],
                                 acc_sh.at[dst_v.at[g * _GROUP + b]],
                                 sem_s, add=True)

        fire_gathers(0, 0)

        def outer(o, _):
            half_step(2 * o, 0)
            half_step(2 * o + 1, 1)
            return ()

        lax.fori_loop(0, ngroups // 2, outer, ())
        drain_scatters(ngroups - 1, 1)
        plsc.subcore_barrier()

        pltpu.sync_copy(acc_sh.at[pl.ds(row0, rows_per_tile)],
                        out_hbm.at[cid, pl.ds(row0, rows_per_tile)])

    return agg_kernel(h, sd3, zeros_pad)


def _silu(x):
    return x * jax.nn.sigmoid(x)


def _tail_body(n_nodes, parts_ref, h_ref, wmsg_ref, wupd_ref,
               state_ref, wstate_ref, lat_ref, wlat_ref,
               w1_ref, b1_ref, w2_ref, b2_ref, o_ref, acc_ref):
    i = pl.program_id(0)
    nblk = pl.num_programs(0)

    @pl.when(i == 0)
    def _():
        acc_ref[...] = jnp.zeros_like(acc_ref)

    agg0 = (parts_ref[0].astype(jnp.float32)
            + parts_ref[1].astype(jnp.float32))
    agg = jnp.dot(agg0, wmsg_ref[...], preferred_element_type=jnp.float32)
    h2 = _silu(jnp.dot(h_ref[...], wupd_ref[...],
                       preferred_element_type=jnp.float32) + agg)
    acc_ref[...] += jnp.sum(h2, axis=0, keepdims=True)

    @pl.when(i == nblk - 1)
    def _():
        cond = (jnp.dot(state_ref[...], wstate_ref[...],
                        preferred_element_type=jnp.float32)
                + jnp.dot(lat_ref[...], wlat_ref[...],
                          preferred_element_type=jnp.float32))
        g = acc_ref[...] / float(n_nodes) + cond
        hid = _silu(jnp.dot(g, w1_ref[...],
                            preferred_element_type=jnp.float32) + b1_ref[...])
        o_ref[...] = jnp.dot(hid, w2_ref[...],
                             preferred_element_type=jnp.float32) + b2_ref[...]


def _tail(parts, h, W_msg, W_upd, state_attr, W_state, lat9, W_lat,
          W1, b1, W2, b2, block_rows):
    n, d = h.shape
    hid = W1.shape[1]
    grid = n // block_rows
    full = lambda shape: pl.BlockSpec(shape, lambda i: tuple(0 for _ in shape))
    return pl.pallas_call(
        functools.partial(_tail_body, n),
        grid=(grid,),
        in_specs=[
            pl.BlockSpec((2, block_rows, d), lambda i: (0, i, 0)),
            pl.BlockSpec((block_rows, d), lambda i: (i, 0)),
            full((d, d)),
            full((d, d)),
            full(state_attr.shape),
            full(W_state.shape),
            full(lat9.shape),
            full(W_lat.shape),
            full((d, hid)),
            full((1, hid)),
            full((hid, 1)),
            full((1, 1)),
        ],
        out_specs=pl.BlockSpec((1, 1), lambda i: (0, 0)),
        out_shape=jax.ShapeDtypeStruct((1, 1), jnp.float32),
        scratch_shapes=[pltpu.VMEM((1, d), jnp.float32)],
    )(parts, h, W_msg, W_upd, state_attr, W_state, lat9, W_lat, W1, b1, W2, b2)


def kernel(node_feat, edge_index, lattice, state_attr, W_in, W_msg, W_upd,
           W_state, W_lat, W1, b1, W2, b2):
    n, d_in = node_feat.shape
    d = W_in.shape[1]
    e = edge_index.shape[1]

    nc, ns = _sc_geometry()
    nw = nc * ns
    chunk = 128
    quantum = nw * chunk * 2 * _GROUP
    e_pad = -(-e // quantum) * quantum
    per_w = e_pad // nw
    chunks = per_w // chunk
    n_pad = -(-(n + 1) // 128) * 128

    src = edge_index[0]
    dst = edge_index[1]
    src3 = jnp.concatenate(
        [src, jnp.zeros((e_pad - e,), jnp.int32)]).reshape(nw, 1, chunks, chunk)
    dst3 = jnp.concatenate(
        [dst, jnp.full((e_pad - e,), n, jnp.int32)]).reshape(nw, 1, chunks, chunk)
    sd3 = jnp.concatenate([src3, dst3], axis=1)
    zeros_pad = jnp.zeros((n_pad, d), jnp.bfloat16)

    h, h_bf = _project(node_feat, W_in, block_rows=2000)
    parts = _sc_aggregate(h_bf, sd3, zeros_pad, n_pad, chunks, chunk)

    out = _tail(parts, h, W_msg, W_upd, state_attr, W_state,
                lattice.reshape(1, 9), W_lat, W1,
                b1.reshape(1, -1), W2, b2.reshape(1, 1), block_rows=2000)
    return out.reshape(1)
